# Initial kernel scaffold; baseline (speedup 1.0000x reference)
import jax
import jax.numpy as jnp
from jax.experimental import pallas as pl


def kernel(x, edge_index, Ws, bs):
    raise NotImplementedError("write your pallas kernel here")


# speedup vs baseline: 10.7668x; 10.7668x over previous
import functools

import jax
import jax.numpy as jnp
from jax import lax
from jax.experimental import pallas as pl
from jax.experimental.pallas import tpu as pltpu
from jax.experimental.pallas import tpu_sc as plsc

N = 10000
E = 320000
BM = 400
GRID = N // BM
EB = 128
NCORES = 2
NSUB = 16
N_PAD = 10240
ROWS_PER_SUB = N_PAD // NSUB
E_PER_CORE = E // NCORES
NBLK_CORE = E_PER_CORE // EB


def _chunking(d):
    if d <= 160:
        dcp = 48 if d == 40 else d
        return 1, dcp
    assert d % 160 == 0
    return d // 160, 160


@functools.lru_cache(maxsize=None)
def _sc_agg(nch, dcp):
    mesh = plsc.VectorSubcoreMesh(core_axis_name="c", subcore_axis_name="s")

    def body(g_hbm, src_hbm, dst_hbm, z_hbm, out_hbm, acc, sidx, didx, rows,
             sem):
        core = lax.axis_index("c")
        sub = lax.axis_index("s")
        r0 = sub * ROWS_PER_SUB
        nblk = 78 + jnp.where(sub < NBLK_CORE - 16 * 78, 1, 0)

        for c in range(nch):
            pltpu.sync_copy(z_hbm.at[pl.ds(r0, ROWS_PER_SUB)],
                            acc.at[pl.ds(r0, ROWS_PER_SUB)])
            plsc.subcore_barrier()

            def eblock(k, _):
                off = core * E_PER_CORE + (sub + k * NSUB) * EB
                pltpu.sync_copy(src_hbm.at[pl.ds(off, EB)], sidx)
                pltpu.sync_copy(dst_hbm.at[pl.ds(off, EB)], didx)
                pltpu.async_copy(g_hbm.at[c].at[sidx], rows, sem).wait()
                pltpu.sync_copy(rows, acc.at[didx], add=True)
                return 0

            lax.fori_loop(0, nblk, eblock, 0)
            plsc.subcore_barrier()
            pltpu.sync_copy(
                acc.at[pl.ds(r0, ROWS_PER_SUB)],
                out_hbm.at[core, c, pl.ds(r0, ROWS_PER_SUB)])
            plsc.subcore_barrier()

    return pl.kernel(
        body,
        out_type=jax.ShapeDtypeStruct((NCORES, nch, N_PAD, dcp), jnp.float32),
        mesh=mesh,
        scratch_types=[
            pltpu.VMEM_SHARED((N_PAD, dcp), jnp.float32),
            pltpu.VMEM((EB,), jnp.int32),
            pltpu.VMEM((EB,), jnp.int32),
            pltpu.VMEM((EB, dcp), jnp.float32),
            pltpu.SemaphoreType.DMA,
        ],
        compiler_params=pltpu.CompilerParams(use_tc_tiling_on_sc=False),
    )


def _sc_run(g, src, dst):
    nch, _N, dcp = g.shape
    z = jnp.zeros((N_PAD, dcp), jnp.float32)
    return _sc_agg(nch, dcp)(g, src, dst, z)


_ROWMAP = lambda i: (i, 0)


def _full(shape):
    return pl.BlockSpec(shape, lambda i: tuple(0 for _ in shape))


def _rows(shape):
    return pl.BlockSpec(shape, lambda i: (i,) + tuple(0 for _ in shape[1:]))


def _gspec(nch, dcp):
    return pl.BlockSpec((nch, BM, dcp), lambda i: (0, i, 0))


def _deg_kernel(p_ref, x_ref, dis_ref, d2_ref, g0_ref):
    deg = p_ref[0, 0, :, 0:1] + p_ref[1, 0, :, 0:1] + 1.0
    dis = lax.rsqrt(deg)
    dis_ref[...] = dis
    d2_ref[...] = dis * dis
    g0_ref[0] = x_ref[...] * dis


def _tc_deg(parts, x):
    return pl.pallas_call(
        _deg_kernel,
        grid=(GRID,),
        in_specs=[
            pl.BlockSpec((2, 1, BM, 16), lambda i: (0, 0, i, 0)),
            _rows((BM, 128)),
        ],
        out_specs=[_rows((BM, 1)), _rows((BM, 1)), _gspec(1, 128)],
        out_shape=[
            jax.ShapeDtypeStruct((N, 1), jnp.float32),
            jax.ShapeDtypeStruct((N, 1), jnp.float32),
            jax.ShapeDtypeStruct((1, N, 128), jnp.float32),
        ],
    )(parts, x)


def _agg_of(s_ref, d):
    nch = s_ref.shape[1]
    parts = [s_ref[0, c] + s_ref[1, c] for c in range(nch)]
    full = parts[0] if nch == 1 else jnp.concatenate(parts, axis=1)
    return full[:, :d]


def _write_g(g_ref, gv, nch, dcp):
    d = gv.shape[1]
    if nch * dcp == d:
        for c in range(nch):
            g_ref[c] = gv[:, c * dcp:(c + 1) * dcp]
    else:
        g_ref[0] = jnp.concatenate(
            [gv, jnp.zeros((gv.shape[0], nch * dcp - d), jnp.float32)], axis=1)


def _tc_mm_proj(h, W, dis, nch, dcp):
    din, dout = W.shape

    def kern(h_ref, w_ref, dis_ref, p_ref, g_ref):
        p = jnp.dot(h_ref[...], w_ref[...],
                    preferred_element_type=jnp.float32,
                    precision=lax.Precision.HIGHEST)
        p_ref[...] = p
        _write_g(g_ref, p * dis_ref[...], nch, dcp)

    return pl.pallas_call(
        kern,
        grid=(GRID,),
        in_specs=[_rows((BM, din)), _full((din, dout)), _rows((BM, 1))],
        out_specs=[_rows((BM, dout)), _gspec(nch, dcp)],
        out_shape=[
            jax.ShapeDtypeStruct((N, dout), jnp.float32),
            jax.ShapeDtypeStruct((nch, N, dcp), jnp.float32),
        ],
    )(h, W, dis)


def _tc_comb(s, p, dis, d2, b, g_out=None):
    dout = p.shape[1]

    def kern(s_ref, p_ref, dis_ref, d2_ref, b_ref, h_ref, *maybe_g):
        agg = _agg_of(s_ref, dout)
        h = jnp.maximum(
            dis_ref[...] * agg + d2_ref[...] * p_ref[...] + b_ref[...], 0.0)
        h_ref[...] = h
        if maybe_g:
            _write_g(maybe_g[0], h * dis_ref[...], *g_out)

    nch_s, dcp_s = s.shape[1], s.shape[3]
    in_specs = [
        pl.BlockSpec((2, nch_s, BM, dcp_s), lambda i: (0, 0, i, 0)),
        _rows((BM, dout)), _rows((BM, 1)), _rows((BM, 1)), _full((1, dout)),
    ]
    out_specs = [_rows((BM, dout))]
    out_shape = [jax.ShapeDtypeStruct((N, dout), jnp.float32)]
    if g_out is not None:
        nch, dcp = g_out
        out_specs.append(_gspec(nch, dcp))
        out_shape.append(jax.ShapeDtypeStruct((nch, N, dcp), jnp.float32))
    res = pl.pallas_call(kern, grid=(GRID,), in_specs=in_specs,
                         out_specs=out_specs, out_shape=out_shape)(
                             s, p, dis, d2, b)
    return res if g_out is not None else (res[0],)


def _tc_mm_agg(s, hprev, dis, d2, W, b, skip=None, g_out=None):
    din, dout = W.shape

    def kern(*refs):
        it = iter(refs)
        s_ref, h_ref, dis_ref, d2_ref, w_ref, b_ref = (
            next(it), next(it), next(it), next(it), next(it), next(it))
        skip_ref = next(it) if skip is not None else None
        o_ref = next(it)
        g_ref = next(it) if g_out is not None else None
        u = dis_ref[...] * _agg_of(s_ref, din) + d2_ref[...] * h_ref[...]
        acc = jnp.dot(u, w_ref[...], preferred_element_type=jnp.float32,
                      precision=lax.Precision.HIGHEST) + b_ref[...]
        if skip_ref is not None:
            acc = acc + skip_ref[...]
        h = jnp.maximum(acc, 0.0)
        o_ref[...] = h
        if g_ref is not None:
            _write_g(g_ref, h * dis_ref[...], *g_out)

    nch_s, dcp_s = s.shape[1], s.shape[3]
    in_specs = [
        pl.BlockSpec((2, nch_s, BM, dcp_s), lambda i: (0, 0, i, 0)),
        _rows((BM, din)), _rows((BM, 1)), _rows((BM, 1)),
        _full((din, dout)), _full((1, dout)),
    ]
    args = [s, hprev, dis, d2, W, b]
    if skip is not None:
        in_specs.append(_rows((BM, dout)))
        args.append(skip)
    out_specs = [_rows((BM, dout))]
    out_shape = [jax.ShapeDtypeStruct((N, dout), jnp.float32)]
    if g_out is not None:
        nch, dcp = g_out
        out_specs.append(_gspec(nch, dcp))
        out_shape.append(jax.ShapeDtypeStruct((nch, N, dcp), jnp.float32))
    res = pl.pallas_call(kern, grid=(GRID,), in_specs=in_specs,
                         out_specs=out_specs, out_shape=out_shape)(*args)
    return res if g_out is not None else (res[0],)


@jax.jit
def _impl(x, edge_index, Ws, bs):
    src = edge_index[0]
    dst = edge_index[1]
    b2d = [b.reshape(1, -1) for b in bs]

    ones16 = jnp.ones((1, N, 16), jnp.float32)
    degp = _sc_run(ones16, src, dst)
    dis, d2, g0 = _tc_deg(degp, x)

    s = _sc_run(g0, src, dst)
    (h1,) = _tc_mm_agg(s, x, dis, d2, Ws[0], b2d[0])

    hs = [h1]
    h = h1
    for i in range(1, 5):
        dout = Ws[i].shape[1]
        nch, dcp = _chunking(dout)
        p, g = _tc_mm_proj(h, Ws[i], dis, nch, dcp)
        s = _sc_run(g, src, dst)
        if i == 4:
            h, g5 = _tc_comb(s, p, dis, d2, b2d[i], g_out=(1, 48))
        else:
            (h,) = _tc_comb(s, p, dis, d2, b2d[i])
            hs.append(h)

    g = g5
    for i in range(5, 9):
        s = _sc_run(g, src, dst)
        skip = hs[8 - i]
        if i < 8:
            nch, dcp = _chunking(Ws[i].shape[1])
            h, g = _tc_mm_agg(s, h, dis, d2, Ws[i], b2d[i], skip=skip,
                              g_out=(nch, dcp))
        else:
            (h,) = _tc_mm_agg(s, h, dis, d2, Ws[i], b2d[i], skip=skip)

    p, g = _tc_mm_proj(h, Ws[9], dis, 1, 128)
    s = _sc_run(g, src, dst)
    (out,) = _tc_comb(s, p, dis, d2, b2d[9])
    return out


def kernel(x, edge_index, Ws, bs):
    return _impl(x, edge_index, list(Ws), list(bs))


# speedup vs baseline: 20.3476x; 1.8898x over previous
import functools

import jax
import jax.numpy as jnp
from jax import lax
from jax.experimental import pallas as pl
from jax.experimental.pallas import tpu as pltpu
from jax.experimental.pallas import tpu_sc as plsc

N = 10000
E = 320000
BM = 400
GRID = N // BM
EB = 128
NCORES = 2
NSUB = 16
N_PAD = 10240
ROWS_PER_SUB = N_PAD // NSUB
E_PER_CORE = E // NCORES
NBLK_CORE = E_PER_CORE // EB


def _chunking(d):
    if d <= 80:
        dcp = 48 if d == 40 else d
        return 1, dcp
    if d == 128:
        return 2, 64
    assert d % 80 == 0
    return d // 80, 80


D_RING = 4
NBMAX = 79
IDXROWS = 88
IDXPAD = 2512


@functools.lru_cache(maxsize=None)
def _sc_agg(nch, dcp):
    mesh = plsc.VectorSubcoreMesh(core_axis_name="c", subcore_axis_name="s")
    n_outer = (NBMAX + D_RING - 1) // D_RING

    def body(g_hbm, src_hbm, dst_hbm, z_hbm, out_hbm, acc, sidx, didx, *rest):
        rows = rest[:D_RING]
        gsems = rest[D_RING:2 * D_RING]
        core = lax.axis_index("c")
        sub = lax.axis_index("s")
        r0 = sub * ROWS_PER_SUB
        nb = 78 + jnp.where(sub < NBLK_CORE - 16 * 78, 1, 0)
        first = core * NBLK_CORE + sub * 78 + jnp.minimum(sub, 2)
        load0 = (first // 8) * 8
        delta = first - load0
        pltpu.sync_copy(src_hbm.at[pl.ds(load0, IDXROWS)], sidx)
        pltpu.sync_copy(dst_hbm.at[pl.ds(load0, IDXROWS)], didx)

        for c in range(nch):
            pltpu.sync_copy(z_hbm.at[pl.ds(r0, ROWS_PER_SUB)],
                            acc.at[pl.ds(r0, ROWS_PER_SUB)])
            plsc.subcore_barrier()

            def fire(slot, b):
                pltpu.async_copy(g_hbm.at[c].at[sidx.at[delta + b]],
                                 rows[slot], gsems[slot])

            for j in range(D_RING):
                fire(j, j)

            def outer(i, _):
                for j in range(D_RING):
                    b = i * D_RING + j

                    @pl.when(b < nb)
                    def _process():
                        pltpu.make_async_copy(
                            g_hbm.at[c].at[sidx.at[delta + b]], rows[j],
                            gsems[j]).wait()
                        pltpu.sync_copy(rows[j], acc.at[didx.at[delta + b]],
                                        add=True)

                    @pl.when(b + D_RING < nb)
                    def _prefetch():
                        fire(j, b + D_RING)

                return 0

            lax.fori_loop(0, n_outer, outer, 0)
            plsc.subcore_barrier()
            pltpu.sync_copy(
                acc.at[pl.ds(r0, ROWS_PER_SUB)],
                out_hbm.at[core, c, pl.ds(r0, ROWS_PER_SUB)])
            plsc.subcore_barrier()

    return pl.kernel(
        body,
        out_type=jax.ShapeDtypeStruct((NCORES, nch, N_PAD, dcp), jnp.float32),
        mesh=mesh,
        scratch_types=[
            pltpu.VMEM_SHARED((N_PAD, dcp), jnp.float32),
            pltpu.VMEM((IDXROWS, EB), jnp.int32),
            pltpu.VMEM((IDXROWS, EB), jnp.int32),
        ] + [pltpu.VMEM((EB, dcp), jnp.float32) for _ in range(D_RING)]
          + [pltpu.SemaphoreType.DMA for _ in range(D_RING)],
        compiler_params=pltpu.CompilerParams(use_tc_tiling_on_sc=False),
    )


def _sc_run(g, src2, dst2):
    nch, _N, dcp = g.shape
    z = jnp.zeros((N_PAD, dcp), jnp.float32)
    return _sc_agg(nch, dcp)(g, src2, dst2, z)


def _blocked_idx(v):
    pad = jnp.zeros((IDXPAD * EB - E,), jnp.int32)
    return jnp.concatenate([v, pad]).reshape(IDXPAD, EB)


_ROWMAP = lambda i: (i, 0)


def _full(shape):
    return pl.BlockSpec(shape, lambda i: tuple(0 for _ in shape))


def _rows(shape):
    return pl.BlockSpec(shape, lambda i: (i,) + tuple(0 for _ in shape[1:]))


def _gspec(nch, dcp):
    return pl.BlockSpec((nch, BM, dcp), lambda i: (0, i, 0))


def _deg_kernel(p_ref, x_ref, dis_ref, d2_ref, g0_ref):
    deg = p_ref[0, 0, :, 0:1] + p_ref[1, 0, :, 0:1] + 1.0
    dis = lax.rsqrt(deg)
    dis_ref[...] = dis
    d2_ref[...] = dis * dis
    _write_g(g0_ref, x_ref[...] * dis, 2, 64)


def _tc_deg(parts, x):
    return pl.pallas_call(
        _deg_kernel,
        grid=(GRID,),
        in_specs=[
            pl.BlockSpec((2, 1, BM, 16), lambda i: (0, 0, i, 0)),
            _rows((BM, 128)),
        ],
        out_specs=[_rows((BM, 1)), _rows((BM, 1)), _gspec(2, 64)],
        out_shape=[
            jax.ShapeDtypeStruct((N, 1), jnp.float32),
            jax.ShapeDtypeStruct((N, 1), jnp.float32),
            jax.ShapeDtypeStruct((2, N, 64), jnp.float32),
        ],
    )(parts, x)


def _agg_of(s_ref, d):
    nch = s_ref.shape[1]
    parts = [s_ref[0, c] + s_ref[1, c] for c in range(nch)]
    full = parts[0] if nch == 1 else jnp.concatenate(parts, axis=1)
    return full[:, :d]


def _write_g(g_ref, gv, nch, dcp):
    d = gv.shape[1]
    if nch * dcp == d:
        for c in range(nch):
            g_ref[c] = gv[:, c * dcp:(c + 1) * dcp]
    else:
        g_ref[0] = jnp.concatenate(
            [gv, jnp.zeros((gv.shape[0], nch * dcp - d), jnp.float32)], axis=1)


def _tc_mm_proj(h, W, dis, nch, dcp):
    din, dout = W.shape

    def kern(h_ref, w_ref, dis_ref, p_ref, g_ref):
        p = jnp.dot(h_ref[...], w_ref[...],
                    preferred_element_type=jnp.float32,
                    precision=lax.Precision.HIGHEST)
        p_ref[...] = p
        _write_g(g_ref, p * dis_ref[...], nch, dcp)

    return pl.pallas_call(
        kern,
        grid=(GRID,),
        in_specs=[_rows((BM, din)), _full((din, dout)), _rows((BM, 1))],
        out_specs=[_rows((BM, dout)), _gspec(nch, dcp)],
        out_shape=[
            jax.ShapeDtypeStruct((N, dout), jnp.float32),
            jax.ShapeDtypeStruct((nch, N, dcp), jnp.float32),
        ],
    )(h, W, dis)


def _tc_comb(s, p, dis, d2, b, g_out=None):
    dout = p.shape[1]

    def kern(s_ref, p_ref, dis_ref, d2_ref, b_ref, h_ref, *maybe_g):
        agg = _agg_of(s_ref, dout)
        h = jnp.maximum(
            dis_ref[...] * agg + d2_ref[...] * p_ref[...] + b_ref[...], 0.0)
        h_ref[...] = h
        if maybe_g:
            _write_g(maybe_g[0], h * dis_ref[...], *g_out)

    nch_s, dcp_s = s.shape[1], s.shape[3]
    in_specs = [
        pl.BlockSpec((2, nch_s, BM, dcp_s), lambda i: (0, 0, i, 0)),
        _rows((BM, dout)), _rows((BM, 1)), _rows((BM, 1)), _full((1, dout)),
    ]
    out_specs = [_rows((BM, dout))]
    out_shape = [jax.ShapeDtypeStruct((N, dout), jnp.float32)]
    if g_out is not None:
        nch, dcp = g_out
        out_specs.append(_gspec(nch, dcp))
        out_shape.append(jax.ShapeDtypeStruct((nch, N, dcp), jnp.float32))
    res = pl.pallas_call(kern, grid=(GRID,), in_specs=in_specs,
                         out_specs=out_specs, out_shape=out_shape)(
                             s, p, dis, d2, b)
    return res if g_out is not None else (res[0],)


def _tc_mm_agg(s, hprev, dis, d2, W, b, skip=None, g_out=None):
    din, dout = W.shape

    def kern(*refs):
        it = iter(refs)
        s_ref, h_ref, dis_ref, d2_ref, w_ref, b_ref = (
            next(it), next(it), next(it), next(it), next(it), next(it))
        skip_ref = next(it) if skip is not None else None
        o_ref = next(it)
        g_ref = next(it) if g_out is not None else None
        u = dis_ref[...] * _agg_of(s_ref, din) + d2_ref[...] * h_ref[...]
        acc = jnp.dot(u, w_ref[...], preferred_element_type=jnp.float32,
                      precision=lax.Precision.HIGHEST) + b_ref[...]
        if skip_ref is not None:
            acc = acc + skip_ref[...]
        h = jnp.maximum(acc, 0.0)
        o_ref[...] = h
        if g_ref is not None:
            _write_g(g_ref, h * dis_ref[...], *g_out)

    nch_s, dcp_s = s.shape[1], s.shape[3]
    in_specs = [
        pl.BlockSpec((2, nch_s, BM, dcp_s), lambda i: (0, 0, i, 0)),
        _rows((BM, din)), _rows((BM, 1)), _rows((BM, 1)),
        _full((din, dout)), _full((1, dout)),
    ]
    args = [s, hprev, dis, d2, W, b]
    if skip is not None:
        in_specs.append(_rows((BM, dout)))
        args.append(skip)
    out_specs = [_rows((BM, dout))]
    out_shape = [jax.ShapeDtypeStruct((N, dout), jnp.float32)]
    if g_out is not None:
        nch, dcp = g_out
        out_specs.append(_gspec(nch, dcp))
        out_shape.append(jax.ShapeDtypeStruct((nch, N, dcp), jnp.float32))
    res = pl.pallas_call(kern, grid=(GRID,), in_specs=in_specs,
                         out_specs=out_specs, out_shape=out_shape)(*args)
    return res if g_out is not None else (res[0],)


@jax.jit
def _impl(x, edge_index, Ws, bs):
    src = _blocked_idx(edge_index[0])
    dst = _blocked_idx(edge_index[1])
    b2d = [b.reshape(1, -1) for b in bs]

    ones16 = jnp.ones((1, N, 16), jnp.float32)
    degp = _sc_run(ones16, src, dst)
    dis, d2, g0 = _tc_deg(degp, x)

    s = _sc_run(g0, src, dst)
    (h1,) = _tc_mm_agg(s, x, dis, d2, Ws[0], b2d[0])

    hs = [h1]
    h = h1
    for i in range(1, 5):
        dout = Ws[i].shape[1]
        nch, dcp = _chunking(dout)
        p, g = _tc_mm_proj(h, Ws[i], dis, nch, dcp)
        s = _sc_run(g, src, dst)
        if i == 4:
            h, g5 = _tc_comb(s, p, dis, d2, b2d[i], g_out=(1, 48))
        else:
            (h,) = _tc_comb(s, p, dis, d2, b2d[i])
            hs.append(h)

    g = g5
    for i in range(5, 9):
        s = _sc_run(g, src, dst)
        skip = hs[8 - i]
        if i < 8:
            nch, dcp = _chunking(Ws[i].shape[1])
            h, g = _tc_mm_agg(s, h, dis, d2, Ws[i], b2d[i], skip=skip,
                              g_out=(nch, dcp))
        else:
            (h,) = _tc_mm_agg(s, h, dis, d2, Ws[i], b2d[i], skip=skip)

    p, g = _tc_mm_proj(h, Ws[9], dis, *_chunking(128))
    s = _sc_run(g, src, dst)
    (out,) = _tc_comb(s, p, dis, d2, b2d[9])
    return out


def kernel(x, edge_index, Ws, bs):
    return _impl(x, edge_index, list(Ws), list(bs))


# speedup vs baseline: 20.3646x; 1.0008x over previous
import functools

import jax
import jax.numpy as jnp
from jax import lax
from jax.experimental import pallas as pl
from jax.experimental.pallas import tpu as pltpu
from jax.experimental.pallas import tpu_sc as plsc

N = 10000
E = 320000
BM = 400
GRID = N // BM
EB = 128
NCORES = 2
NSUB = 16
N_PAD = 10240
ROWS_PER_SUB = N_PAD // NSUB
E_PER_CORE = E // NCORES
NBLK_CORE = E_PER_CORE // EB


def _chunking(d):
    if d <= 80:
        dcp = 48 if d == 40 else d
        return 1, dcp
    if d == 128:
        return 2, 64
    assert d % 80 == 0
    return d // 80, 80


D_RING = 4
NBMAX = 79
IDXROWS = 88
IDXPAD = 2512


@functools.lru_cache(maxsize=None)
def _sc_agg(nch, dcp):
    mesh = plsc.VectorSubcoreMesh(core_axis_name="c", subcore_axis_name="s")
    n_outer = (NBMAX + D_RING - 1) // D_RING

    def body(g_hbm, src_hbm, dst_hbm, z_hbm, out_hbm, acc, sidx, didx, *rest):
        rows = rest[:D_RING]
        gsems = rest[D_RING:2 * D_RING]
        ssems = rest[2 * D_RING:3 * D_RING]
        core = lax.axis_index("c")
        sub = lax.axis_index("s")
        r0 = sub * ROWS_PER_SUB
        nb = 78 + jnp.where(sub < NBLK_CORE - 16 * 78, 1, 0)
        first = core * NBLK_CORE + sub * 78 + jnp.minimum(sub, 2)
        load0 = (first // 8) * 8
        delta = first - load0
        pltpu.sync_copy(src_hbm.at[pl.ds(load0, IDXROWS)], sidx)
        pltpu.sync_copy(dst_hbm.at[pl.ds(load0, IDXROWS)], didx)

        for c in range(nch):
            pltpu.sync_copy(z_hbm.at[pl.ds(r0, ROWS_PER_SUB)],
                            acc.at[pl.ds(r0, ROWS_PER_SUB)])
            plsc.subcore_barrier()

            def fire(slot, b):
                pltpu.async_copy(g_hbm.at[c].at[sidx.at[delta + b]],
                                 rows[slot], gsems[slot])

            for j in range(D_RING):
                fire(j, j)

            def outer(i, _):
                for j in range(D_RING):
                    b = i * D_RING + j

                    @pl.when(b < nb)
                    def _process():
                        pltpu.make_async_copy(
                            g_hbm.at[c].at[sidx.at[delta + b]], rows[j],
                            gsems[j]).wait()
                        pltpu.async_copy(rows[j], acc.at[didx.at[delta + b]],
                                         ssems[j], add=True)

                    @pl.when(b + D_RING < nb)
                    def _prefetch():
                        pltpu.make_async_copy(
                            rows[j], acc.at[didx.at[delta + b]],
                            ssems[j]).wait()
                        fire(j, b + D_RING)

                return 0

            lax.fori_loop(0, n_outer, outer, 0)
            for j in range(D_RING):
                pltpu.make_async_copy(rows[j], acc.at[didx.at[delta]],
                                      ssems[j]).wait()
            plsc.subcore_barrier()
            pltpu.sync_copy(
                acc.at[pl.ds(r0, ROWS_PER_SUB)],
                out_hbm.at[core, c, pl.ds(r0, ROWS_PER_SUB)])
            plsc.subcore_barrier()

    return pl.kernel(
        body,
        out_type=jax.ShapeDtypeStruct((NCORES, nch, N_PAD, dcp), jnp.float32),
        mesh=mesh,
        scratch_types=[
            pltpu.VMEM_SHARED((N_PAD, dcp), jnp.float32),
            pltpu.VMEM((IDXROWS, EB), jnp.int32),
            pltpu.VMEM((IDXROWS, EB), jnp.int32),
        ] + [pltpu.VMEM((EB, dcp), jnp.float32) for _ in range(D_RING)]
          + [pltpu.SemaphoreType.DMA for _ in range(2 * D_RING)],
        compiler_params=pltpu.CompilerParams(use_tc_tiling_on_sc=False),
    )


def _sc_run(g, src2, dst2):
    nch, _N, dcp = g.shape
    z = jnp.zeros((N_PAD, dcp), jnp.float32)
    return _sc_agg(nch, dcp)(g, src2, dst2, z)


def _blocked_idx(v):
    pad = jnp.zeros((IDXPAD * EB - E,), jnp.int32)
    return jnp.concatenate([v, pad]).reshape(IDXPAD, EB)


_ROWMAP = lambda i: (i, 0)


def _full(shape):
    return pl.BlockSpec(shape, lambda i: tuple(0 for _ in shape))


def _rows(shape):
    return pl.BlockSpec(shape, lambda i: (i,) + tuple(0 for _ in shape[1:]))


def _gspec(nch, dcp):
    return pl.BlockSpec((nch, BM, dcp), lambda i: (0, i, 0))


def _deg_kernel(p_ref, x_ref, dis_ref, d2_ref, g0_ref):
    deg = p_ref[0, 0, :, 0:1] + p_ref[1, 0, :, 0:1] + 1.0
    dis = lax.rsqrt(deg)
    dis_ref[...] = dis
    d2_ref[...] = dis * dis
    _write_g(g0_ref, x_ref[...] * dis, 2, 64)


def _tc_deg(parts, x):
    return pl.pallas_call(
        _deg_kernel,
        grid=(GRID,),
        in_specs=[
            pl.BlockSpec((2, 1, BM, 16), lambda i: (0, 0, i, 0)),
            _rows((BM, 128)),
        ],
        out_specs=[_rows((BM, 1)), _rows((BM, 1)), _gspec(2, 64)],
        out_shape=[
            jax.ShapeDtypeStruct((N, 1), jnp.float32),
            jax.ShapeDtypeStruct((N, 1), jnp.float32),
            jax.ShapeDtypeStruct((2, N, 64), jnp.float32),
        ],
    )(parts, x)


def _agg_of(s_ref, d):
    nch = s_ref.shape[1]
    parts = [s_ref[0, c] + s_ref[1, c] for c in range(nch)]
    full = parts[0] if nch == 1 else jnp.concatenate(parts, axis=1)
    return full[:, :d]


def _write_g(g_ref, gv, nch, dcp):
    d = gv.shape[1]
    if nch * dcp == d:
        for c in range(nch):
            g_ref[c] = gv[:, c * dcp:(c + 1) * dcp]
    else:
        g_ref[0] = jnp.concatenate(
            [gv, jnp.zeros((gv.shape[0], nch * dcp - d), jnp.float32)], axis=1)


def _tc_mm_proj(h, W, dis, nch, dcp):
    din, dout = W.shape

    def kern(h_ref, w_ref, dis_ref, p_ref, g_ref):
        p = jnp.dot(h_ref[...], w_ref[...],
                    preferred_element_type=jnp.float32,
                    precision=lax.Precision.HIGHEST)
        p_ref[...] = p
        _write_g(g_ref, p * dis_ref[...], nch, dcp)

    return pl.pallas_call(
        kern,
        grid=(GRID,),
        in_specs=[_rows((BM, din)), _full((din, dout)), _rows((BM, 1))],
        out_specs=[_rows((BM, dout)), _gspec(nch, dcp)],
        out_shape=[
            jax.ShapeDtypeStruct((N, dout), jnp.float32),
            jax.ShapeDtypeStruct((nch, N, dcp), jnp.float32),
        ],
    )(h, W, dis)


def _tc_comb(s, p, dis, d2, b, g_out=None):
    dout = p.shape[1]

    def kern(s_ref, p_ref, dis_ref, d2_ref, b_ref, h_ref, *maybe_g):
        agg = _agg_of(s_ref, dout)
        h = jnp.maximum(
            dis_ref[...] * agg + d2_ref[...] * p_ref[...] + b_ref[...], 0.0)
        h_ref[...] = h
        if maybe_g:
            _write_g(maybe_g[0], h * dis_ref[...], *g_out)

    nch_s, dcp_s = s.shape[1], s.shape[3]
    in_specs = [
        pl.BlockSpec((2, nch_s, BM, dcp_s), lambda i: (0, 0, i, 0)),
        _rows((BM, dout)), _rows((BM, 1)), _rows((BM, 1)), _full((1, dout)),
    ]
    out_specs = [_rows((BM, dout))]
    out_shape = [jax.ShapeDtypeStruct((N, dout), jnp.float32)]
    if g_out is not None:
        nch, dcp = g_out
        out_specs.append(_gspec(nch, dcp))
        out_shape.append(jax.ShapeDtypeStruct((nch, N, dcp), jnp.float32))
    res = pl.pallas_call(kern, grid=(GRID,), in_specs=in_specs,
                         out_specs=out_specs, out_shape=out_shape)(
                             s, p, dis, d2, b)
    return res if g_out is not None else (res[0],)


def _tc_mm_agg(s, hprev, dis, d2, W, b, skip=None, g_out=None):
    din, dout = W.shape

    def kern(*refs):
        it = iter(refs)
        s_ref, h_ref, dis_ref, d2_ref, w_ref, b_ref = (
            next(it), next(it), next(it), next(it), next(it), next(it))
        skip_ref = next(it) if skip is not None else None
        o_ref = next(it)
        g_ref = next(it) if g_out is not None else None
        u = dis_ref[...] * _agg_of(s_ref, din) + d2_ref[...] * h_ref[...]
        acc = jnp.dot(u, w_ref[...], preferred_element_type=jnp.float32,
                      precision=lax.Precision.HIGHEST) + b_ref[...]
        if skip_ref is not None:
            acc = acc + skip_ref[...]
        h = jnp.maximum(acc, 0.0)
        o_ref[...] = h
        if g_ref is not None:
            _write_g(g_ref, h * dis_ref[...], *g_out)

    nch_s, dcp_s = s.shape[1], s.shape[3]
    in_specs = [
        pl.BlockSpec((2, nch_s, BM, dcp_s), lambda i: (0, 0, i, 0)),
        _rows((BM, din)), _rows((BM, 1)), _rows((BM, 1)),
        _full((din, dout)), _full((1, dout)),
    ]
    args = [s, hprev, dis, d2, W, b]
    if skip is not None:
        in_specs.append(_rows((BM, dout)))
        args.append(skip)
    out_specs = [_rows((BM, dout))]
    out_shape = [jax.ShapeDtypeStruct((N, dout), jnp.float32)]
    if g_out is not None:
        nch, dcp = g_out
        out_specs.append(_gspec(nch, dcp))
        out_shape.append(jax.ShapeDtypeStruct((nch, N, dcp), jnp.float32))
    res = pl.pallas_call(kern, grid=(GRID,), in_specs=in_specs,
                         out_specs=out_specs, out_shape=out_shape)(*args)
    return res if g_out is not None else (res[0],)


@jax.jit
def _impl(x, edge_index, Ws, bs):
    src = _blocked_idx(edge_index[0])
    dst = _blocked_idx(edge_index[1])
    b2d = [b.reshape(1, -1) for b in bs]

    ones16 = jnp.ones((1, N, 16), jnp.float32)
    degp = _sc_run(ones16, src, dst)
    dis, d2, g0 = _tc_deg(degp, x)

    s = _sc_run(g0, src, dst)
    (h1,) = _tc_mm_agg(s, x, dis, d2, Ws[0], b2d[0])

    hs = [h1]
    h = h1
    for i in range(1, 5):
        dout = Ws[i].shape[1]
        nch, dcp = _chunking(dout)
        p, g = _tc_mm_proj(h, Ws[i], dis, nch, dcp)
        s = _sc_run(g, src, dst)
        if i == 4:
            h, g5 = _tc_comb(s, p, dis, d2, b2d[i], g_out=(1, 48))
        else:
            (h,) = _tc_comb(s, p, dis, d2, b2d[i])
            hs.append(h)

    g = g5
    for i in range(5, 9):
        s = _sc_run(g, src, dst)
        skip = hs[8 - i]
        if i < 8:
            nch, dcp = _chunking(Ws[i].shape[1])
            h, g = _tc_mm_agg(s, h, dis, d2, Ws[i], b2d[i], skip=skip,
                              g_out=(nch, dcp))
        else:
            (h,) = _tc_mm_agg(s, h, dis, d2, Ws[i], b2d[i], skip=skip)

    p, g = _tc_mm_proj(h, Ws[9], dis, *_chunking(128))
    s = _sc_run(g, src, dst)
    (out,) = _tc_comb(s, p, dis, d2, b2d[9])
    return out


def kernel(x, edge_index, Ws, bs):
    return _impl(x, edge_index, list(Ws), list(bs))


# speedup vs baseline: 21.2120x; 1.0416x over previous
import functools

import jax
import jax.numpy as jnp
from jax import lax
from jax.experimental import pallas as pl
from jax.experimental.pallas import tpu as pltpu
from jax.experimental.pallas import tpu_sc as plsc

N = 10000
E = 320000
BM = 400
GRID = N // BM
EB = 128
NCORES = 2
NSUB = 16
N_PAD = 10240
ROWS_PER_SUB = N_PAD // NSUB
E_PER_CORE = E // NCORES
NBLK_CORE = E_PER_CORE // EB


def _chunking(d):
    if d <= 80:
        dcp = 48 if d == 40 else d
        return 1, dcp
    if d == 128:
        return 2, 64
    assert d % 80 == 0
    return d // 80, 80


D_RING = 4
NBMAX = 79
IDXROWS = 88
IDXPAD = 2512


@functools.lru_cache(maxsize=None)
def _sc_agg(nch, dcp):
    mesh = plsc.VectorSubcoreMesh(core_axis_name="c", subcore_axis_name="s")
    n_outer = (NBMAX + D_RING - 1) // D_RING

    def body(g_hbm, src_hbm, dst_hbm, z_hbm, out_hbm, acc, sidx, didx, *rest):
        rows = rest[:D_RING]
        gsems = rest[D_RING:2 * D_RING]
        ssems = rest[2 * D_RING:3 * D_RING]
        core = lax.axis_index("c")
        sub = lax.axis_index("s")
        r0 = sub * ROWS_PER_SUB
        nb = 78 + jnp.where(sub < NBLK_CORE - 16 * 78, 1, 0)
        first = core * NBLK_CORE + sub * 78 + jnp.minimum(sub, 2)
        load0 = (first // 8) * 8
        delta = first - load0
        pltpu.sync_copy(src_hbm.at[pl.ds(load0, IDXROWS)], sidx)
        pltpu.sync_copy(dst_hbm.at[pl.ds(load0, IDXROWS)], didx)

        for c in range(nch):
            pltpu.sync_copy(z_hbm.at[pl.ds(r0, ROWS_PER_SUB)],
                            acc.at[pl.ds(r0, ROWS_PER_SUB)])
            plsc.subcore_barrier()

            def fire(slot, b):
                pltpu.async_copy(g_hbm.at[c].at[sidx.at[delta + b]],
                                 rows[slot], gsems[slot])

            for j in range(D_RING):
                fire(j, j)

            def outer(i, _):
                for j in range(D_RING):
                    b = i * D_RING + j

                    @pl.when(b < nb)
                    def _process():
                        pltpu.make_async_copy(
                            g_hbm.at[c].at[sidx.at[delta + b]], rows[j],
                            gsems[j]).wait()
                        pltpu.async_copy(rows[j], acc.at[didx.at[delta + b]],
                                         ssems[j], add=True)

                    @pl.when(b + D_RING < nb)
                    def _prefetch():
                        pltpu.make_async_copy(
                            rows[j], acc.at[didx.at[delta + b]],
                            ssems[j]).wait()
                        fire(j, b + D_RING)

                return 0

            lax.fori_loop(0, n_outer, outer, 0)
            for j in range(D_RING):
                pltpu.make_async_copy(rows[j], acc.at[didx.at[delta]],
                                      ssems[j]).wait()
            plsc.subcore_barrier()
            pltpu.sync_copy(
                acc.at[pl.ds(r0, ROWS_PER_SUB)],
                out_hbm.at[core, c, pl.ds(r0, ROWS_PER_SUB)])
            plsc.subcore_barrier()

    return pl.kernel(
        body,
        out_type=jax.ShapeDtypeStruct((NCORES, nch, N_PAD, dcp), jnp.float32),
        mesh=mesh,
        scratch_types=[
            pltpu.VMEM_SHARED((N_PAD, dcp), jnp.float32),
            pltpu.VMEM((IDXROWS, EB), jnp.int32),
            pltpu.VMEM((IDXROWS, EB), jnp.int32),
        ] + [pltpu.VMEM((EB, dcp), jnp.float32) for _ in range(D_RING)]
          + [pltpu.SemaphoreType.DMA for _ in range(2 * D_RING)],
        compiler_params=pltpu.CompilerParams(use_tc_tiling_on_sc=False),
    )


def _sc_run(g, src2, dst2):
    nch, _N, dcp = g.shape
    z = jnp.zeros((N_PAD, dcp), jnp.float32)
    return _sc_agg(nch, dcp)(g, src2, dst2, z)


def _blocked_idx(v):
    pad = jnp.zeros((IDXPAD * EB - E,), jnp.int32)
    return jnp.concatenate([v, pad]).reshape(IDXPAD, EB)


_ROWMAP = lambda i: (i, 0)


def _full(shape):
    return pl.BlockSpec(shape, lambda i: tuple(0 for _ in shape))


def _rows(shape):
    return pl.BlockSpec(shape, lambda i: (i,) + tuple(0 for _ in shape[1:]))


def _gspec(nch, dcp):
    return pl.BlockSpec((nch, BM, dcp), lambda i: (0, i, 0))


def _deg_kernel(p_ref, x_ref, dis_ref, d2_ref, g0_ref):
    deg = p_ref[0, 0, :, 0:1] + p_ref[1, 0, :, 0:1] + 1.0
    dis = lax.rsqrt(deg)
    dis_ref[...] = dis
    d2_ref[...] = dis * dis
    _write_g(g0_ref, x_ref[...] * dis, 2, 64)


def _tc_deg(parts, x):
    return pl.pallas_call(
        _deg_kernel,
        grid=(GRID,),
        in_specs=[
            pl.BlockSpec((2, 1, BM, 16), lambda i: (0, 0, i, 0)),
            _rows((BM, 128)),
        ],
        out_specs=[_rows((BM, 1)), _rows((BM, 1)), _gspec(2, 64)],
        out_shape=[
            jax.ShapeDtypeStruct((N, 1), jnp.float32),
            jax.ShapeDtypeStruct((N, 1), jnp.float32),
            jax.ShapeDtypeStruct((2, N, 64), jnp.float32),
        ],
    )(parts, x)


def _agg_of(s_ref, d):
    nch = s_ref.shape[1]
    parts = [s_ref[0, c] + s_ref[1, c] for c in range(nch)]
    full = parts[0] if nch == 1 else jnp.concatenate(parts, axis=1)
    return full[:, :d]


def _write_g(g_ref, gv, nch, dcp):
    d = gv.shape[1]
    if nch * dcp == d:
        for c in range(nch):
            g_ref[c] = gv[:, c * dcp:(c + 1) * dcp]
    else:
        g_ref[0] = jnp.concatenate(
            [gv, jnp.zeros((gv.shape[0], nch * dcp - d), jnp.float32)], axis=1)


def _dot(a, b):
    return jnp.dot(a, b, preferred_element_type=jnp.float32,
                   precision=lax.Precision.HIGHEST)


def _tc_agg_proj(s, hbase, dis, d2, W, b, Wn, nch, dcp, skip=None):
    din, dout = W.shape
    dnext = Wn.shape[1]

    def kern(*refs):
        it = iter(refs)
        s_ref, hb_ref, dis_ref, d2_ref, w_ref, b_ref, wn_ref = (
            next(it) for _ in range(7))
        skip_ref = next(it) if skip is not None else None
        h_ref, p_ref, g_ref = next(it), next(it), next(it)
        u = dis_ref[...] * _agg_of(s_ref, din) + d2_ref[...] * hb_ref[...]
        acc = _dot(u, w_ref[...]) + b_ref[...]
        if skip_ref is not None:
            acc = acc + skip_ref[...]
        h = jnp.maximum(acc, 0.0)
        h_ref[...] = h
        p = _dot(h, wn_ref[...])
        p_ref[...] = p
        _write_g(g_ref, p * dis_ref[...], nch, dcp)

    nch_s, dcp_s = s.shape[1], s.shape[3]
    in_specs = [
        pl.BlockSpec((2, nch_s, BM, dcp_s), lambda i: (0, 0, i, 0)),
        _rows((BM, din)), _rows((BM, 1)), _rows((BM, 1)),
        _full((din, dout)), _full((1, dout)), _full((dout, dnext)),
    ]
    args = [s, hbase, dis, d2, W, b, Wn]
    if skip is not None:
        in_specs.append(_rows((BM, dout)))
        args.append(skip)
    return pl.pallas_call(
        kern,
        grid=(GRID,),
        in_specs=in_specs,
        out_specs=[_rows((BM, dout)), _rows((BM, dnext)), _gspec(nch, dcp)],
        out_shape=[
            jax.ShapeDtypeStruct((N, dout), jnp.float32),
            jax.ShapeDtypeStruct((N, dnext), jnp.float32),
            jax.ShapeDtypeStruct((nch, N, dcp), jnp.float32),
        ],
    )(*args)


def _tc_comb_proj(s, p, dis, d2, b, Wn, nch, dcp):
    dout = p.shape[1]
    dnext = Wn.shape[1]

    def kern(s_ref, p_ref, dis_ref, d2_ref, b_ref, wn_ref, h_ref, pn_ref,
             g_ref):
        agg = _agg_of(s_ref, dout)
        h = jnp.maximum(
            dis_ref[...] * agg + d2_ref[...] * p_ref[...] + b_ref[...], 0.0)
        h_ref[...] = h
        pn = _dot(h, wn_ref[...])
        pn_ref[...] = pn
        _write_g(g_ref, pn * dis_ref[...], nch, dcp)

    nch_s, dcp_s = s.shape[1], s.shape[3]
    return pl.pallas_call(
        kern,
        grid=(GRID,),
        in_specs=[
            pl.BlockSpec((2, nch_s, BM, dcp_s), lambda i: (0, 0, i, 0)),
            _rows((BM, dout)), _rows((BM, 1)), _rows((BM, 1)),
            _full((1, dout)), _full((dout, dnext)),
        ],
        out_specs=[_rows((BM, dout)), _rows((BM, dnext)), _gspec(nch, dcp)],
        out_shape=[
            jax.ShapeDtypeStruct((N, dout), jnp.float32),
            jax.ShapeDtypeStruct((N, dnext), jnp.float32),
            jax.ShapeDtypeStruct((nch, N, dcp), jnp.float32),
        ],
    )(s, p, dis, d2, b, Wn)


def _tc_comb(s, p, dis, d2, b, g_out=None):
    dout = p.shape[1]

    def kern(s_ref, p_ref, dis_ref, d2_ref, b_ref, h_ref, *maybe_g):
        agg = _agg_of(s_ref, dout)
        h = jnp.maximum(
            dis_ref[...] * agg + d2_ref[...] * p_ref[...] + b_ref[...], 0.0)
        h_ref[...] = h
        if maybe_g:
            _write_g(maybe_g[0], h * dis_ref[...], *g_out)

    nch_s, dcp_s = s.shape[1], s.shape[3]
    in_specs = [
        pl.BlockSpec((2, nch_s, BM, dcp_s), lambda i: (0, 0, i, 0)),
        _rows((BM, dout)), _rows((BM, 1)), _rows((BM, 1)), _full((1, dout)),
    ]
    out_specs = [_rows((BM, dout))]
    out_shape = [jax.ShapeDtypeStruct((N, dout), jnp.float32)]
    if g_out is not None:
        nch, dcp = g_out
        out_specs.append(_gspec(nch, dcp))
        out_shape.append(jax.ShapeDtypeStruct((nch, N, dcp), jnp.float32))
    res = pl.pallas_call(kern, grid=(GRID,), in_specs=in_specs,
                         out_specs=out_specs, out_shape=out_shape)(
                             s, p, dis, d2, b)
    return res if g_out is not None else (res[0],)


def _tc_mm_agg(s, hprev, dis, d2, W, b, skip=None, g_out=None):
    din, dout = W.shape

    def kern(*refs):
        it = iter(refs)
        s_ref, h_ref, dis_ref, d2_ref, w_ref, b_ref = (
            next(it), next(it), next(it), next(it), next(it), next(it))
        skip_ref = next(it) if skip is not None else None
        o_ref = next(it)
        g_ref = next(it) if g_out is not None else None
        u = dis_ref[...] * _agg_of(s_ref, din) + d2_ref[...] * h_ref[...]
        acc = jnp.dot(u, w_ref[...], preferred_element_type=jnp.float32,
                      precision=lax.Precision.HIGHEST) + b_ref[...]
        if skip_ref is not None:
            acc = acc + skip_ref[...]
        h = jnp.maximum(acc, 0.0)
        o_ref[...] = h
        if g_ref is not None:
            _write_g(g_ref, h * dis_ref[...], *g_out)

    nch_s, dcp_s = s.shape[1], s.shape[3]
    in_specs = [
        pl.BlockSpec((2, nch_s, BM, dcp_s), lambda i: (0, 0, i, 0)),
        _rows((BM, din)), _rows((BM, 1)), _rows((BM, 1)),
        _full((din, dout)), _full((1, dout)),
    ]
    args = [s, hprev, dis, d2, W, b]
    if skip is not None:
        in_specs.append(_rows((BM, dout)))
        args.append(skip)
    out_specs = [_rows((BM, dout))]
    out_shape = [jax.ShapeDtypeStruct((N, dout), jnp.float32)]
    if g_out is not None:
        nch, dcp = g_out
        out_specs.append(_gspec(nch, dcp))
        out_shape.append(jax.ShapeDtypeStruct((nch, N, dcp), jnp.float32))
    res = pl.pallas_call(kern, grid=(GRID,), in_specs=in_specs,
                         out_specs=out_specs, out_shape=out_shape)(*args)
    return res if g_out is not None else (res[0],)


@jax.jit
def _impl(x, edge_index, Ws, bs):
    src = _blocked_idx(edge_index[0])
    dst = _blocked_idx(edge_index[1])
    b2d = [b.reshape(1, -1) for b in bs]

    ones16 = jnp.ones((1, N, 16), jnp.float32)
    degp = _sc_run(ones16, src, dst)
    dis, d2, g0 = _tc_deg(degp, x)

    s = _sc_run(g0, src, dst)
    h1, p1, g1 = _tc_agg_proj(s, x, dis, d2, Ws[0], b2d[0], Ws[1],
                              *_chunking(320))

    s = _sc_run(g1, src, dst)
    h2, p2, g2 = _tc_comb_proj(s, p1, dis, d2, b2d[1], Ws[2], *_chunking(160))
    s = _sc_run(g2, src, dst)
    h3, p3, g3 = _tc_comb_proj(s, p2, dis, d2, b2d[2], Ws[3], *_chunking(80))
    s = _sc_run(g3, src, dst)
    h4, p4, g4 = _tc_comb_proj(s, p3, dis, d2, b2d[3], Ws[4], *_chunking(40))

    s = _sc_run(g4, src, dst)
    h5, g5 = _tc_comb(s, p4, dis, d2, b2d[4], g_out=(1, 48))

    s = _sc_run(g5, src, dst)
    h6, g6 = _tc_mm_agg(s, h5, dis, d2, Ws[5], b2d[5], skip=h4,
                        g_out=_chunking(80))
    s = _sc_run(g6, src, dst)
    h7, g7 = _tc_mm_agg(s, h6, dis, d2, Ws[6], b2d[6], skip=h3,
                        g_out=_chunking(160))
    s = _sc_run(g7, src, dst)
    h8, g8 = _tc_mm_agg(s, h7, dis, d2, Ws[7], b2d[7], skip=h2,
                        g_out=_chunking(320))

    s = _sc_run(g8, src, dst)
    _h9, p9, g9 = _tc_agg_proj(s, h8, dis, d2, Ws[8], b2d[8], Ws[9],
                               *_chunking(128), skip=h1)

    s = _sc_run(g9, src, dst)
    (out,) = _tc_comb(s, p9, dis, d2, b2d[9])
    return out


def kernel(x, edge_index, Ws, bs):
    return _impl(x, edge_index, list(Ws), list(bs))


# speedup vs baseline: 22.2828x; 1.0505x over previous
import functools

import jax
import jax.numpy as jnp
from jax import lax
from jax.experimental import pallas as pl
from jax.experimental.pallas import tpu as pltpu
from jax.experimental.pallas import tpu_sc as plsc

N = 10000
E = 320000
BM = 400
GRID = N // BM
EB = 128
NCORES = 2
NSUB = 16
N_PAD = 10240
ROWS_PER_SUB = N_PAD // NSUB
E_PER_CORE = E // NCORES
NBLK_CORE = E_PER_CORE // EB


def _chunking(d):
    if d <= 80:
        dcp = 48 if d == 40 else d
        return 1, dcp
    if d == 128:
        return 2, 64
    assert d % 80 == 0
    return d // 80, 80


D_RING = 4
NBMAX = 79
IDXROWS = 88
IDXPAD = 2512


@functools.lru_cache(maxsize=None)
def _sc_agg(nch, dcp):
    mesh = plsc.VectorSubcoreMesh(core_axis_name="c", subcore_axis_name="s")
    n_outer = (NBMAX + D_RING - 1) // D_RING

    def body(g_hbm, src_hbm, dst_hbm, z_hbm, out_hbm, acc, sidx, didx, *rest):
        rows = rest[:D_RING]
        gsems = rest[D_RING:2 * D_RING]
        ssems = rest[2 * D_RING:3 * D_RING]
        core = lax.axis_index("c")
        sub = lax.axis_index("s")
        r0 = sub * ROWS_PER_SUB
        nb = 78 + jnp.where(sub < NBLK_CORE - 16 * 78, 1, 0)
        first = core * NBLK_CORE + sub * 78 + jnp.minimum(sub, 2)
        load0 = (first // 8) * 8
        delta = first - load0
        pltpu.sync_copy(src_hbm.at[pl.ds(load0, IDXROWS)], sidx)
        pltpu.sync_copy(dst_hbm.at[pl.ds(load0, IDXROWS)], didx)

        for c in range(nch):
            pltpu.sync_copy(z_hbm.at[pl.ds(r0, ROWS_PER_SUB)],
                            acc.at[pl.ds(r0, ROWS_PER_SUB)])
            plsc.subcore_barrier()

            def fire(slot, b):
                pltpu.async_copy(g_hbm.at[c].at[sidx.at[delta + b]],
                                 rows[slot], gsems[slot])

            for j in range(D_RING):
                fire(j, j)

            def outer(i, _):
                for j in range(D_RING):
                    b = i * D_RING + j

                    @pl.when(b < nb)
                    def _process():
                        pltpu.make_async_copy(
                            g_hbm.at[c].at[sidx.at[delta + b]], rows[j],
                            gsems[j]).wait()
                        pltpu.async_copy(rows[j], acc.at[didx.at[delta + b]],
                                         ssems[j], add=True)

                    @pl.when(b + D_RING < nb)
                    def _prefetch():
                        pltpu.make_async_copy(
                            rows[j], acc.at[didx.at[delta + b]],
                            ssems[j]).wait()
                        fire(j, b + D_RING)

                return 0

            lax.fori_loop(0, n_outer, outer, 0)
            for j in range(D_RING):
                pltpu.make_async_copy(rows[j], acc.at[didx.at[delta]],
                                      ssems[j]).wait()
            plsc.subcore_barrier()
            pltpu.sync_copy(
                acc.at[pl.ds(r0, ROWS_PER_SUB)],
                out_hbm.at[core, c, pl.ds(r0, ROWS_PER_SUB)])
            plsc.subcore_barrier()

    return pl.kernel(
        body,
        out_type=jax.ShapeDtypeStruct((NCORES, nch, N_PAD, dcp), jnp.float32),
        mesh=mesh,
        scratch_types=[
            pltpu.VMEM_SHARED((N_PAD, dcp), jnp.float32),
            pltpu.VMEM((IDXROWS, EB), jnp.int32),
            pltpu.VMEM((IDXROWS, EB), jnp.int32),
        ] + [pltpu.VMEM((EB, dcp), jnp.float32) for _ in range(D_RING)]
          + [pltpu.SemaphoreType.DMA for _ in range(2 * D_RING)],
        compiler_params=pltpu.CompilerParams(use_tc_tiling_on_sc=False),
    )


def _sc_run(g, src2, dst2):
    nch, _N, dcp = g.shape
    z = jnp.zeros((N_PAD, dcp), jnp.float32)
    return _sc_agg(nch, dcp)(g, src2, dst2, z)


def _blocked_idx(v):
    pad = jnp.zeros((IDXPAD * EB - E,), jnp.int32)
    return jnp.concatenate([v, pad]).reshape(IDXPAD, EB)


_ROWMAP = lambda i: (i, 0)


def _full(shape):
    return pl.BlockSpec(shape, lambda i: tuple(0 for _ in shape))


def _rows(shape):
    return pl.BlockSpec(shape, lambda i: (i,) + tuple(0 for _ in shape[1:]))


def _gspec(nch, dcp):
    return pl.BlockSpec((nch, BM, dcp), lambda i: (0, i, 0))


def _deg_kernel(p_ref, x_ref, dis_ref, d2_ref, g0_ref):
    deg = p_ref[0, 0, :, 0:1] + p_ref[1, 0, :, 0:1] + 1.0
    dis = lax.rsqrt(deg)
    dis_ref[...] = dis
    d2_ref[...] = dis * dis
    _write_g(g0_ref, x_ref[...] * dis, 2, 64)


def _tc_deg(parts, x):
    return pl.pallas_call(
        _deg_kernel,
        grid=(GRID,),
        in_specs=[
            pl.BlockSpec((2, 1, BM, 16), lambda i: (0, 0, i, 0)),
            _rows((BM, 128)),
        ],
        out_specs=[_rows((BM, 1)), _rows((BM, 1)), _gspec(2, 64)],
        out_shape=[
            jax.ShapeDtypeStruct((N, 1), jnp.float32),
            jax.ShapeDtypeStruct((N, 1), jnp.float32),
            jax.ShapeDtypeStruct((2, N, 64), jnp.float32),
        ],
    )(parts, x)


def _agg_of(s_ref, d):
    nch = s_ref.shape[1]
    parts = [s_ref[0, c] + s_ref[1, c] for c in range(nch)]
    full = parts[0] if nch == 1 else jnp.concatenate(parts, axis=1)
    return full[:, :d]


def _write_g(g_ref, gv, nch, dcp):
    d = gv.shape[1]
    if nch * dcp == d:
        for c in range(nch):
            g_ref[c] = gv[:, c * dcp:(c + 1) * dcp]
    else:
        g_ref[0] = jnp.concatenate(
            [gv, jnp.zeros((gv.shape[0], nch * dcp - d), jnp.float32)], axis=1)


def _dot(a, b):
    return jnp.dot(a, b, preferred_element_type=jnp.float32,
                   precision=lax.Precision.DEFAULT)


def _tc_agg_proj(s, hbase, dis, d2, W, b, Wn, nch, dcp, skip=None):
    din, dout = W.shape
    dnext = Wn.shape[1]

    def kern(*refs):
        it = iter(refs)
        s_ref, hb_ref, dis_ref, d2_ref, w_ref, b_ref, wn_ref = (
            next(it) for _ in range(7))
        skip_ref = next(it) if skip is not None else None
        h_ref, p_ref, g_ref = next(it), next(it), next(it)
        u = dis_ref[...] * _agg_of(s_ref, din) + d2_ref[...] * hb_ref[...]
        acc = _dot(u, w_ref[...]) + b_ref[...]
        if skip_ref is not None:
            acc = acc + skip_ref[...]
        h = jnp.maximum(acc, 0.0)
        h_ref[...] = h
        p = _dot(h, wn_ref[...])
        p_ref[...] = p
        _write_g(g_ref, p * dis_ref[...], nch, dcp)

    nch_s, dcp_s = s.shape[1], s.shape[3]
    in_specs = [
        pl.BlockSpec((2, nch_s, BM, dcp_s), lambda i: (0, 0, i, 0)),
        _rows((BM, din)), _rows((BM, 1)), _rows((BM, 1)),
        _full((din, dout)), _full((1, dout)), _full((dout, dnext)),
    ]
    args = [s, hbase, dis, d2, W, b, Wn]
    if skip is not None:
        in_specs.append(_rows((BM, dout)))
        args.append(skip)
    return pl.pallas_call(
        kern,
        grid=(GRID,),
        in_specs=in_specs,
        out_specs=[_rows((BM, dout)), _rows((BM, dnext)), _gspec(nch, dcp)],
        out_shape=[
            jax.ShapeDtypeStruct((N, dout), jnp.float32),
            jax.ShapeDtypeStruct((N, dnext), jnp.float32),
            jax.ShapeDtypeStruct((nch, N, dcp), jnp.float32),
        ],
    )(*args)


def _tc_comb_proj(s, p, dis, d2, b, Wn, nch, dcp):
    dout = p.shape[1]
    dnext = Wn.shape[1]

    def kern(s_ref, p_ref, dis_ref, d2_ref, b_ref, wn_ref, h_ref, pn_ref,
             g_ref):
        agg = _agg_of(s_ref, dout)
        h = jnp.maximum(
            dis_ref[...] * agg + d2_ref[...] * p_ref[...] + b_ref[...], 0.0)
        h_ref[...] = h
        pn = _dot(h, wn_ref[...])
        pn_ref[...] = pn
        _write_g(g_ref, pn * dis_ref[...], nch, dcp)

    nch_s, dcp_s = s.shape[1], s.shape[3]
    return pl.pallas_call(
        kern,
        grid=(GRID,),
        in_specs=[
            pl.BlockSpec((2, nch_s, BM, dcp_s), lambda i: (0, 0, i, 0)),
            _rows((BM, dout)), _rows((BM, 1)), _rows((BM, 1)),
            _full((1, dout)), _full((dout, dnext)),
        ],
        out_specs=[_rows((BM, dout)), _rows((BM, dnext)), _gspec(nch, dcp)],
        out_shape=[
            jax.ShapeDtypeStruct((N, dout), jnp.float32),
            jax.ShapeDtypeStruct((N, dnext), jnp.float32),
            jax.ShapeDtypeStruct((nch, N, dcp), jnp.float32),
        ],
    )(s, p, dis, d2, b, Wn)


def _tc_comb(s, p, dis, d2, b, g_out=None):
    dout = p.shape[1]

    def kern(s_ref, p_ref, dis_ref, d2_ref, b_ref, h_ref, *maybe_g):
        agg = _agg_of(s_ref, dout)
        h = jnp.maximum(
            dis_ref[...] * agg + d2_ref[...] * p_ref[...] + b_ref[...], 0.0)
        h_ref[...] = h
        if maybe_g:
            _write_g(maybe_g[0], h * dis_ref[...], *g_out)

    nch_s, dcp_s = s.shape[1], s.shape[3]
    in_specs = [
        pl.BlockSpec((2, nch_s, BM, dcp_s), lambda i: (0, 0, i, 0)),
        _rows((BM, dout)), _rows((BM, 1)), _rows((BM, 1)), _full((1, dout)),
    ]
    out_specs = [_rows((BM, dout))]
    out_shape = [jax.ShapeDtypeStruct((N, dout), jnp.float32)]
    if g_out is not None:
        nch, dcp = g_out
        out_specs.append(_gspec(nch, dcp))
        out_shape.append(jax.ShapeDtypeStruct((nch, N, dcp), jnp.float32))
    res = pl.pallas_call(kern, grid=(GRID,), in_specs=in_specs,
                         out_specs=out_specs, out_shape=out_shape)(
                             s, p, dis, d2, b)
    return res if g_out is not None else (res[0],)


def _tc_mm_agg(s, hprev, dis, d2, W, b, skip=None, g_out=None):
    din, dout = W.shape

    def kern(*refs):
        it = iter(refs)
        s_ref, h_ref, dis_ref, d2_ref, w_ref, b_ref = (
            next(it), next(it), next(it), next(it), next(it), next(it))
        skip_ref = next(it) if skip is not None else None
        o_ref = next(it)
        g_ref = next(it) if g_out is not None else None
        u = dis_ref[...] * _agg_of(s_ref, din) + d2_ref[...] * h_ref[...]
        acc = jnp.dot(u, w_ref[...], preferred_element_type=jnp.float32,
                      precision=lax.Precision.DEFAULT) + b_ref[...]
        if skip_ref is not None:
            acc = acc + skip_ref[...]
        h = jnp.maximum(acc, 0.0)
        o_ref[...] = h
        if g_ref is not None:
            _write_g(g_ref, h * dis_ref[...], *g_out)

    nch_s, dcp_s = s.shape[1], s.shape[3]
    in_specs = [
        pl.BlockSpec((2, nch_s, BM, dcp_s), lambda i: (0, 0, i, 0)),
        _rows((BM, din)), _rows((BM, 1)), _rows((BM, 1)),
        _full((din, dout)), _full((1, dout)),
    ]
    args = [s, hprev, dis, d2, W, b]
    if skip is not None:
        in_specs.append(_rows((BM, dout)))
        args.append(skip)
    out_specs = [_rows((BM, dout))]
    out_shape = [jax.ShapeDtypeStruct((N, dout), jnp.float32)]
    if g_out is not None:
        nch, dcp = g_out
        out_specs.append(_gspec(nch, dcp))
        out_shape.append(jax.ShapeDtypeStruct((nch, N, dcp), jnp.float32))
    res = pl.pallas_call(kern, grid=(GRID,), in_specs=in_specs,
                         out_specs=out_specs, out_shape=out_shape)(*args)
    return res if g_out is not None else (res[0],)


@jax.jit
def _impl(x, edge_index, Ws, bs):
    src = _blocked_idx(edge_index[0])
    dst = _blocked_idx(edge_index[1])
    b2d = [b.reshape(1, -1) for b in bs]

    ones16 = jnp.ones((1, N, 16), jnp.float32)
    degp = _sc_run(ones16, src, dst)
    dis, d2, g0 = _tc_deg(degp, x)

    s = _sc_run(g0, src, dst)
    h1, p1, g1 = _tc_agg_proj(s, x, dis, d2, Ws[0], b2d[0], Ws[1],
                              *_chunking(320))

    s = _sc_run(g1, src, dst)
    h2, p2, g2 = _tc_comb_proj(s, p1, dis, d2, b2d[1], Ws[2], *_chunking(160))
    s = _sc_run(g2, src, dst)
    h3, p3, g3 = _tc_comb_proj(s, p2, dis, d2, b2d[2], Ws[3], *_chunking(80))
    s = _sc_run(g3, src, dst)
    h4, p4, g4 = _tc_comb_proj(s, p3, dis, d2, b2d[3], Ws[4], *_chunking(40))

    s = _sc_run(g4, src, dst)
    h5, g5 = _tc_comb(s, p4, dis, d2, b2d[4], g_out=(1, 48))

    s = _sc_run(g5, src, dst)
    h6, g6 = _tc_mm_agg(s, h5, dis, d2, Ws[5], b2d[5], skip=h4,
                        g_out=_chunking(80))
    s = _sc_run(g6, src, dst)
    h7, g7 = _tc_mm_agg(s, h6, dis, d2, Ws[6], b2d[6], skip=h3,
                        g_out=_chunking(160))
    s = _sc_run(g7, src, dst)
    h8, g8 = _tc_mm_agg(s, h7, dis, d2, Ws[7], b2d[7], skip=h2,
                        g_out=_chunking(320))

    s = _sc_run(g8, src, dst)
    _h9, p9, g9 = _tc_agg_proj(s, h8, dis, d2, Ws[8], b2d[8], Ws[9],
                               *_chunking(128), skip=h1)

    s = _sc_run(g9, src, dst)
    (out,) = _tc_comb(s, p9, dis, d2, b2d[9])
    return out


def kernel(x, edge_index, Ws, bs):
    return _impl(x, edge_index, list(Ws), list(bs))


# speedup vs baseline: 24.2614x; 1.0888x over previous
import functools

import jax
import jax.numpy as jnp
from jax import lax
from jax.experimental import pallas as pl
from jax.experimental.pallas import tpu as pltpu
from jax.experimental.pallas import tpu_sc as plsc

N = 10000
E = 320000
BM = 400
GRID = N // BM
EB = 128
NCORES = 2
NSUB = 16
N_PAD = 10240
ROWS_PER_SUB = N_PAD // NSUB
E_PER_CORE = E // NCORES
NBLK_CORE = E_PER_CORE // EB


def _chunking(d):
    if d <= 80:
        dcp = 48 if d == 40 else d
        return 1, dcp
    if d == 128:
        return 2, 64
    assert d % 80 == 0
    return d // 80, 80


D_RING = 4
NBMAX = 79
IDXROWS = 88
IDXPAD = 2512


@functools.lru_cache(maxsize=None)
def _sc_agg(nch, dcp):
    fsplit = nch >= 2
    d_ring = 3 if fsplit else D_RING
    mesh = plsc.VectorSubcoreMesh(core_axis_name="c", subcore_axis_name="s")
    if fsplit:
        assert nch % NCORES == 0
        nblk_tot = 2 * NBLK_CORE
        base_blk = nblk_tot // NSUB
        nbmax = base_blk + 1
        out_shape = (nch, N_PAD, dcp)
        n_my_chunks = nch // NCORES
    else:
        nblk_tot = NBLK_CORE
        base_blk = NBLK_CORE // NSUB
        nbmax = base_blk + 1
        out_shape = (NCORES, nch, N_PAD, dcp)
        n_my_chunks = 1
    idxrows = ((nbmax + 7) // 8) * 8 + 8
    n_outer = (nbmax + d_ring - 1) // d_ring
    rem = nblk_tot - NSUB * base_blk

    def body(g_hbm, src_hbm, dst_hbm, z_hbm, out_hbm, acc, sidx, didx, *rest):
        rows = rest[:d_ring]
        gsems = rest[d_ring:2 * d_ring]
        ssems = rest[2 * d_ring:3 * d_ring]
        core = lax.axis_index("c")
        sub = lax.axis_index("s")
        r0 = sub * ROWS_PER_SUB
        nb = base_blk + jnp.where(sub < rem, 1, 0)
        first = sub * base_blk + jnp.minimum(sub, rem)
        if not fsplit:
            first = core * NBLK_CORE + first
        load0 = (first // 8) * 8
        delta = first - load0
        pltpu.sync_copy(src_hbm.at[pl.ds(load0, idxrows)], sidx)
        pltpu.sync_copy(dst_hbm.at[pl.ds(load0, idxrows)], didx)

        for ci in range(n_my_chunks):
            c = core * 0 + ci if not fsplit else core + NCORES * ci
            pltpu.sync_copy(z_hbm.at[pl.ds(r0, ROWS_PER_SUB)],
                            acc.at[pl.ds(r0, ROWS_PER_SUB)])
            plsc.subcore_barrier()

            g_c = g_hbm.at[ci] if not fsplit else g_hbm.at[c]

            def fire(slot, b):
                pltpu.async_copy(g_c.at[sidx.at[delta + b]],
                                 rows[slot], gsems[slot])

            for j in range(d_ring):
                fire(j, j)

            def outer(i, _):
                for j in range(d_ring):
                    b = i * d_ring + j

                    @pl.when(b < nb)
                    def _process():
                        pltpu.make_async_copy(
                            g_c.at[sidx.at[delta + b]], rows[j],
                            gsems[j]).wait()
                        pltpu.async_copy(rows[j], acc.at[didx.at[delta + b]],
                                         ssems[j], add=True)

                    @pl.when(b + d_ring < nb)
                    def _prefetch():
                        pltpu.make_async_copy(
                            rows[j], acc.at[didx.at[delta + b]],
                            ssems[j]).wait()
                        fire(j, b + d_ring)

                return 0

            lax.fori_loop(0, n_outer, outer, 0)
            for j in range(d_ring):
                pltpu.make_async_copy(rows[j], acc.at[didx.at[delta]],
                                      ssems[j]).wait()
            plsc.subcore_barrier()
            if fsplit:
                dst_slice = out_hbm.at[c, pl.ds(r0, ROWS_PER_SUB)]
            else:
                dst_slice = out_hbm.at[core, ci, pl.ds(r0, ROWS_PER_SUB)]
            pltpu.sync_copy(acc.at[pl.ds(r0, ROWS_PER_SUB)], dst_slice)
            plsc.subcore_barrier()

    return pl.kernel(
        body,
        out_type=jax.ShapeDtypeStruct(out_shape, jnp.float32),
        mesh=mesh,
        scratch_types=[
            pltpu.VMEM_SHARED((N_PAD, dcp), jnp.float32),
            pltpu.VMEM((idxrows, EB), jnp.int32),
            pltpu.VMEM((idxrows, EB), jnp.int32),
        ] + [pltpu.VMEM((EB, dcp), jnp.float32) for _ in range(d_ring)]
          + [pltpu.SemaphoreType.DMA for _ in range(2 * d_ring)],
        compiler_params=pltpu.CompilerParams(use_tc_tiling_on_sc=False),
    )


def _sc_run(g, src2, dst2):
    nch, _N, dcp = g.shape
    z = jnp.zeros((N_PAD, dcp), jnp.float32)
    return _sc_agg(nch, dcp)(g, src2, dst2, z)


def _blocked_idx(v):
    pad = jnp.zeros((IDXPAD * EB - E,), jnp.int32)
    return jnp.concatenate([v, pad]).reshape(IDXPAD, EB)


_ROWMAP = lambda i: (i, 0)


def _full(shape):
    return pl.BlockSpec(shape, lambda i: tuple(0 for _ in shape))


def _rows(shape):
    return pl.BlockSpec(shape, lambda i: (i,) + tuple(0 for _ in shape[1:]))


def _gspec(nch, dcp):
    return pl.BlockSpec((nch, BM, dcp), lambda i: (0, i, 0))


def _sspec(s):
    if s.ndim == 4:
        return pl.BlockSpec((2, s.shape[1], BM, s.shape[3]),
                            lambda i: (0, 0, i, 0))
    return pl.BlockSpec((s.shape[0], BM, s.shape[2]), lambda i: (0, i, 0))


def _deg_kernel(p_ref, x_ref, dis_ref, d2_ref, g0_ref):
    deg = p_ref[0, 0, :, 0:1] + p_ref[1, 0, :, 0:1] + 1.0
    dis = lax.rsqrt(deg)
    dis_ref[...] = dis
    d2_ref[...] = dis * dis
    _write_g(g0_ref, x_ref[...] * dis, 2, 64)


def _tc_deg(parts, x):
    return pl.pallas_call(
        _deg_kernel,
        grid=(GRID,),
        in_specs=[
            pl.BlockSpec((2, 1, BM, 16), lambda i: (0, 0, i, 0)),
            _rows((BM, 128)),
        ],
        out_specs=[_rows((BM, 1)), _rows((BM, 1)), _gspec(2, 64)],
        out_shape=[
            jax.ShapeDtypeStruct((N, 1), jnp.float32),
            jax.ShapeDtypeStruct((N, 1), jnp.float32),
            jax.ShapeDtypeStruct((2, N, 64), jnp.float32),
        ],
    )(parts, x)


def _agg_of(s_ref, d):
    if len(s_ref.shape) == 4:
        nch = s_ref.shape[1]
        parts = [s_ref[0, c] + s_ref[1, c] for c in range(nch)]
    else:
        nch = s_ref.shape[0]
        parts = [s_ref[c] for c in range(nch)]
    full = parts[0] if nch == 1 else jnp.concatenate(parts, axis=1)
    return full[:, :d]


def _write_g(g_ref, gv, nch, dcp):
    d = gv.shape[1]
    if nch * dcp == d:
        for c in range(nch):
            g_ref[c] = gv[:, c * dcp:(c + 1) * dcp]
    else:
        g_ref[0] = jnp.concatenate(
            [gv, jnp.zeros((gv.shape[0], nch * dcp - d), jnp.float32)], axis=1)


def _dot(a, b):
    return jnp.dot(a, b, preferred_element_type=jnp.float32,
                   precision=lax.Precision.DEFAULT)


def _tc_agg_proj(s, hbase, dis, d2, W, b, Wn, nch, dcp, skip=None):
    din, dout = W.shape
    dnext = Wn.shape[1]

    def kern(*refs):
        it = iter(refs)
        s_ref, hb_ref, dis_ref, d2_ref, w_ref, b_ref, wn_ref = (
            next(it) for _ in range(7))
        skip_ref = next(it) if skip is not None else None
        h_ref, p_ref, g_ref = next(it), next(it), next(it)
        u = dis_ref[...] * _agg_of(s_ref, din) + d2_ref[...] * hb_ref[...]
        acc = _dot(u, w_ref[...]) + b_ref[...]
        if skip_ref is not None:
            acc = acc + skip_ref[...]
        h = jnp.maximum(acc, 0.0)
        h_ref[...] = h
        p = _dot(h, wn_ref[...])
        p_ref[...] = p
        _write_g(g_ref, p * dis_ref[...], nch, dcp)

    in_specs = [
        _sspec(s),
        _rows((BM, din)), _rows((BM, 1)), _rows((BM, 1)),
        _full((din, dout)), _full((1, dout)), _full((dout, dnext)),
    ]
    args = [s, hbase, dis, d2, W, b, Wn]
    if skip is not None:
        in_specs.append(_rows((BM, dout)))
        args.append(skip)
    return pl.pallas_call(
        kern,
        grid=(GRID,),
        in_specs=in_specs,
        out_specs=[_rows((BM, dout)), _rows((BM, dnext)), _gspec(nch, dcp)],
        out_shape=[
            jax.ShapeDtypeStruct((N, dout), jnp.float32),
            jax.ShapeDtypeStruct((N, dnext), jnp.float32),
            jax.ShapeDtypeStruct((nch, N, dcp), jnp.float32),
        ],
    )(*args)


def _tc_comb_proj(s, p, dis, d2, b, Wn, nch, dcp):
    dout = p.shape[1]
    dnext = Wn.shape[1]

    def kern(s_ref, p_ref, dis_ref, d2_ref, b_ref, wn_ref, h_ref, pn_ref,
             g_ref):
        agg = _agg_of(s_ref, dout)
        h = jnp.maximum(
            dis_ref[...] * agg + d2_ref[...] * p_ref[...] + b_ref[...], 0.0)
        h_ref[...] = h
        pn = _dot(h, wn_ref[...])
        pn_ref[...] = pn
        _write_g(g_ref, pn * dis_ref[...], nch, dcp)

    return pl.pallas_call(
        kern,
        grid=(GRID,),
        in_specs=[
            _sspec(s),
            _rows((BM, dout)), _rows((BM, 1)), _rows((BM, 1)),
            _full((1, dout)), _full((dout, dnext)),
        ],
        out_specs=[_rows((BM, dout)), _rows((BM, dnext)), _gspec(nch, dcp)],
        out_shape=[
            jax.ShapeDtypeStruct((N, dout), jnp.float32),
            jax.ShapeDtypeStruct((N, dnext), jnp.float32),
            jax.ShapeDtypeStruct((nch, N, dcp), jnp.float32),
        ],
    )(s, p, dis, d2, b, Wn)


def _tc_comb(s, p, dis, d2, b, g_out=None):
    dout = p.shape[1]

    def kern(s_ref, p_ref, dis_ref, d2_ref, b_ref, h_ref, *maybe_g):
        agg = _agg_of(s_ref, dout)
        h = jnp.maximum(
            dis_ref[...] * agg + d2_ref[...] * p_ref[...] + b_ref[...], 0.0)
        h_ref[...] = h
        if maybe_g:
            _write_g(maybe_g[0], h * dis_ref[...], *g_out)

    in_specs = [
        _sspec(s),
        _rows((BM, dout)), _rows((BM, 1)), _rows((BM, 1)), _full((1, dout)),
    ]
    out_specs = [_rows((BM, dout))]
    out_shape = [jax.ShapeDtypeStruct((N, dout), jnp.float32)]
    if g_out is not None:
        nch, dcp = g_out
        out_specs.append(_gspec(nch, dcp))
        out_shape.append(jax.ShapeDtypeStruct((nch, N, dcp), jnp.float32))
    res = pl.pallas_call(kern, grid=(GRID,), in_specs=in_specs,
                         out_specs=out_specs, out_shape=out_shape)(
                             s, p, dis, d2, b)
    return res if g_out is not None else (res[0],)


def _tc_mm_agg(s, hprev, dis, d2, W, b, skip=None, g_out=None):
    din, dout = W.shape

    def kern(*refs):
        it = iter(refs)
        s_ref, h_ref, dis_ref, d2_ref, w_ref, b_ref = (
            next(it), next(it), next(it), next(it), next(it), next(it))
        skip_ref = next(it) if skip is not None else None
        o_ref = next(it)
        g_ref = next(it) if g_out is not None else None
        u = dis_ref[...] * _agg_of(s_ref, din) + d2_ref[...] * h_ref[...]
        acc = jnp.dot(u, w_ref[...], preferred_element_type=jnp.float32,
                      precision=lax.Precision.DEFAULT) + b_ref[...]
        if skip_ref is not None:
            acc = acc + skip_ref[...]
        h = jnp.maximum(acc, 0.0)
        o_ref[...] = h
        if g_ref is not None:
            _write_g(g_ref, h * dis_ref[...], *g_out)

    in_specs = [
        _sspec(s),
        _rows((BM, din)), _rows((BM, 1)), _rows((BM, 1)),
        _full((din, dout)), _full((1, dout)),
    ]
    args = [s, hprev, dis, d2, W, b]
    if skip is not None:
        in_specs.append(_rows((BM, dout)))
        args.append(skip)
    out_specs = [_rows((BM, dout))]
    out_shape = [jax.ShapeDtypeStruct((N, dout), jnp.float32)]
    if g_out is not None:
        nch, dcp = g_out
        out_specs.append(_gspec(nch, dcp))
        out_shape.append(jax.ShapeDtypeStruct((nch, N, dcp), jnp.float32))
    res = pl.pallas_call(kern, grid=(GRID,), in_specs=in_specs,
                         out_specs=out_specs, out_shape=out_shape)(*args)
    return res if g_out is not None else (res[0],)


@jax.jit
def _impl(x, edge_index, Ws, bs):
    src = _blocked_idx(edge_index[0])
    dst = _blocked_idx(edge_index[1])
    b2d = [b.reshape(1, -1) for b in bs]

    ones16 = jnp.ones((1, N, 16), jnp.float32)
    degp = _sc_run(ones16, src, dst)
    dis, d2, g0 = _tc_deg(degp, x)

    s = _sc_run(g0, src, dst)
    h1, p1, g1 = _tc_agg_proj(s, x, dis, d2, Ws[0], b2d[0], Ws[1],
                              *_chunking(320))

    s = _sc_run(g1, src, dst)
    h2, p2, g2 = _tc_comb_proj(s, p1, dis, d2, b2d[1], Ws[2], *_chunking(160))
    s = _sc_run(g2, src, dst)
    h3, p3, g3 = _tc_comb_proj(s, p2, dis, d2, b2d[2], Ws[3], *_chunking(80))
    s = _sc_run(g3, src, dst)
    h4, p4, g4 = _tc_comb_proj(s, p3, dis, d2, b2d[3], Ws[4], *_chunking(40))

    s = _sc_run(g4, src, dst)
    h5, g5 = _tc_comb(s, p4, dis, d2, b2d[4], g_out=(1, 48))

    s = _sc_run(g5, src, dst)
    h6, g6 = _tc_mm_agg(s, h5, dis, d2, Ws[5], b2d[5], skip=h4,
                        g_out=_chunking(80))
    s = _sc_run(g6, src, dst)
    h7, g7 = _tc_mm_agg(s, h6, dis, d2, Ws[6], b2d[6], skip=h3,
                        g_out=_chunking(160))
    s = _sc_run(g7, src, dst)
    h8, g8 = _tc_mm_agg(s, h7, dis, d2, Ws[7], b2d[7], skip=h2,
                        g_out=_chunking(320))

    s = _sc_run(g8, src, dst)
    _h9, p9, g9 = _tc_agg_proj(s, h8, dis, d2, Ws[8], b2d[8], Ws[9],
                               *_chunking(128), skip=h1)

    s = _sc_run(g9, src, dst)
    (out,) = _tc_comb(s, p9, dis, d2, b2d[9])
    return out


def kernel(x, edge_index, Ws, bs):
    return _impl(x, edge_index, list(Ws), list(bs))


# speedup vs baseline: 24.3543x; 1.0038x over previous
import functools

import jax
import jax.numpy as jnp
from jax import lax
from jax.experimental import pallas as pl
from jax.experimental.pallas import tpu as pltpu
from jax.experimental.pallas import tpu_sc as plsc

N = 10000
E = 320000
BM = 400
GRID = N // BM
EB = 128
NCORES = 2
NSUB = 16
N_PAD = 10240
ROWS_PER_SUB = N_PAD // NSUB
E_PER_CORE = E // NCORES
NBLK_CORE = E_PER_CORE // EB


def _chunking(d):
    if d <= 80:
        dcp = 48 if d == 40 else d
        return 1, dcp
    if d == 128:
        return 2, 64
    assert d % 80 == 0
    return d // 80, 80


D_RING = 4
NBMAX = 79
IDXROWS = 88
IDXPAD = 2512


@functools.lru_cache(maxsize=None)
def _sc_agg(nch, dcp):
    fsplit = nch >= 2
    d_ring = 3 if fsplit else 5
    mesh = plsc.VectorSubcoreMesh(core_axis_name="c", subcore_axis_name="s")
    if fsplit:
        assert nch % NCORES == 0
        nblk_tot = 2 * NBLK_CORE
        base_blk = nblk_tot // NSUB
        nbmax = base_blk + 1
        out_shape = (nch, N_PAD, dcp)
        n_my_chunks = nch // NCORES
    else:
        nblk_tot = NBLK_CORE
        base_blk = NBLK_CORE // NSUB
        nbmax = base_blk + 1
        out_shape = (NCORES, nch, N_PAD, dcp)
        n_my_chunks = 1
    idxrows = ((nbmax + 7) // 8) * 8 + 8
    n_outer = (nbmax + d_ring - 1) // d_ring
    rem = nblk_tot - NSUB * base_blk

    def body(g_hbm, src_hbm, dst_hbm, z_hbm, out_hbm, acc, sidx, didx, *rest):
        rows = rest[:d_ring]
        gsems = rest[d_ring:2 * d_ring]
        ssems = rest[2 * d_ring:3 * d_ring]
        core = lax.axis_index("c")
        sub = lax.axis_index("s")
        r0 = sub * ROWS_PER_SUB
        nb = base_blk + jnp.where(sub < rem, 1, 0)
        first = sub * base_blk + jnp.minimum(sub, rem)
        if not fsplit:
            first = core * NBLK_CORE + first
        load0 = (first // 8) * 8
        delta = first - load0
        pltpu.sync_copy(src_hbm.at[pl.ds(load0, idxrows)], sidx)
        pltpu.sync_copy(dst_hbm.at[pl.ds(load0, idxrows)], didx)

        for ci in range(n_my_chunks):
            c = core * 0 + ci if not fsplit else core + NCORES * ci
            pltpu.sync_copy(z_hbm.at[pl.ds(r0, ROWS_PER_SUB)],
                            acc.at[pl.ds(r0, ROWS_PER_SUB)])
            plsc.subcore_barrier()

            g_c = g_hbm.at[ci] if not fsplit else g_hbm.at[c]

            def fire(slot, b):
                pltpu.async_copy(g_c.at[sidx.at[delta + b]],
                                 rows[slot], gsems[slot])

            for j in range(d_ring):
                fire(j, j)

            def outer(i, _):
                for j in range(d_ring):
                    b = i * d_ring + j

                    @pl.when(b < nb)
                    def _process():
                        pltpu.make_async_copy(
                            g_c.at[sidx.at[delta + b]], rows[j],
                            gsems[j]).wait()
                        pltpu.async_copy(rows[j], acc.at[didx.at[delta + b]],
                                         ssems[j], add=True)

                    @pl.when(b + d_ring < nb)
                    def _prefetch():
                        pltpu.make_async_copy(
                            rows[j], acc.at[didx.at[delta + b]],
                            ssems[j]).wait()
                        fire(j, b + d_ring)

                return 0

            lax.fori_loop(0, n_outer, outer, 0)
            for j in range(d_ring):
                pltpu.make_async_copy(rows[j], acc.at[didx.at[delta]],
                                      ssems[j]).wait()
            plsc.subcore_barrier()
            if fsplit:
                dst_slice = out_hbm.at[c, pl.ds(r0, ROWS_PER_SUB)]
            else:
                dst_slice = out_hbm.at[core, ci, pl.ds(r0, ROWS_PER_SUB)]
            pltpu.sync_copy(acc.at[pl.ds(r0, ROWS_PER_SUB)], dst_slice)
            plsc.subcore_barrier()

    return pl.kernel(
        body,
        out_type=jax.ShapeDtypeStruct(out_shape, jnp.float32),
        mesh=mesh,
        scratch_types=[
            pltpu.VMEM_SHARED((N_PAD, dcp), jnp.float32),
            pltpu.VMEM((idxrows, EB), jnp.int32),
            pltpu.VMEM((idxrows, EB), jnp.int32),
        ] + [pltpu.VMEM((EB, dcp), jnp.float32) for _ in range(d_ring)]
          + [pltpu.SemaphoreType.DMA for _ in range(2 * d_ring)],
        compiler_params=pltpu.CompilerParams(use_tc_tiling_on_sc=False),
    )


def _sc_run(g, src2, dst2):
    nch, _N, dcp = g.shape
    z = jnp.zeros((N_PAD, dcp), jnp.float32)
    return _sc_agg(nch, dcp)(g, src2, dst2, z)


def _blocked_idx(v):
    pad = jnp.zeros((IDXPAD * EB - E,), jnp.int32)
    return jnp.concatenate([v, pad]).reshape(IDXPAD, EB)


_ROWMAP = lambda i: (i, 0)


def _full(shape):
    return pl.BlockSpec(shape, lambda i: tuple(0 for _ in shape))


def _rows(shape):
    return pl.BlockSpec(shape, lambda i: (i,) + tuple(0 for _ in shape[1:]))


def _gspec(nch, dcp):
    return pl.BlockSpec((nch, BM, dcp), lambda i: (0, i, 0))


def _sspec(s):
    if s.ndim == 4:
        return pl.BlockSpec((2, s.shape[1], BM, s.shape[3]),
                            lambda i: (0, 0, i, 0))
    return pl.BlockSpec((s.shape[0], BM, s.shape[2]), lambda i: (0, i, 0))


def _deg_kernel(p_ref, x_ref, dis_ref, d2_ref, g0_ref):
    deg = p_ref[0, 0, :, 0:1] + p_ref[1, 0, :, 0:1] + 1.0
    dis = lax.rsqrt(deg)
    dis_ref[...] = dis
    d2_ref[...] = dis * dis
    _write_g(g0_ref, x_ref[...] * dis, 2, 64)


def _tc_deg(parts, x):
    return pl.pallas_call(
        _deg_kernel,
        grid=(GRID,),
        in_specs=[
            pl.BlockSpec((2, 1, BM, 16), lambda i: (0, 0, i, 0)),
            _rows((BM, 128)),
        ],
        out_specs=[_rows((BM, 1)), _rows((BM, 1)), _gspec(2, 64)],
        out_shape=[
            jax.ShapeDtypeStruct((N, 1), jnp.float32),
            jax.ShapeDtypeStruct((N, 1), jnp.float32),
            jax.ShapeDtypeStruct((2, N, 64), jnp.float32),
        ],
    )(parts, x)


def _agg_of(s_ref, d):
    if len(s_ref.shape) == 4:
        nch = s_ref.shape[1]
        parts = [s_ref[0, c] + s_ref[1, c] for c in range(nch)]
    else:
        nch = s_ref.shape[0]
        parts = [s_ref[c] for c in range(nch)]
    full = parts[0] if nch == 1 else jnp.concatenate(parts, axis=1)
    return full[:, :d]


def _write_g(g_ref, gv, nch, dcp):
    d = gv.shape[1]
    if nch * dcp == d:
        for c in range(nch):
            g_ref[c] = gv[:, c * dcp:(c + 1) * dcp]
    else:
        g_ref[0] = jnp.concatenate(
            [gv, jnp.zeros((gv.shape[0], nch * dcp - d), jnp.float32)], axis=1)


def _dot(a, b):
    return jnp.dot(a, b, preferred_element_type=jnp.float32,
                   precision=lax.Precision.DEFAULT)


def _tc_agg_proj(s, hbase, dis, d2, W, b, Wn, nch, dcp, skip=None):
    din, dout = W.shape
    dnext = Wn.shape[1]

    def kern(*refs):
        it = iter(refs)
        s_ref, hb_ref, dis_ref, d2_ref, w_ref, b_ref, wn_ref = (
            next(it) for _ in range(7))
        skip_ref = next(it) if skip is not None else None
        h_ref, p_ref, g_ref = next(it), next(it), next(it)
        u = dis_ref[...] * _agg_of(s_ref, din) + d2_ref[...] * hb_ref[...]
        acc = _dot(u, w_ref[...]) + b_ref[...]
        if skip_ref is not None:
            acc = acc + skip_ref[...]
        h = jnp.maximum(acc, 0.0)
        h_ref[...] = h
        p = _dot(h, wn_ref[...])
        p_ref[...] = p
        _write_g(g_ref, p * dis_ref[...], nch, dcp)

    in_specs = [
        _sspec(s),
        _rows((BM, din)), _rows((BM, 1)), _rows((BM, 1)),
        _full((din, dout)), _full((1, dout)), _full((dout, dnext)),
    ]
    args = [s, hbase, dis, d2, W, b, Wn]
    if skip is not None:
        in_specs.append(_rows((BM, dout)))
        args.append(skip)
    return pl.pallas_call(
        kern,
        grid=(GRID,),
        in_specs=in_specs,
        out_specs=[_rows((BM, dout)), _rows((BM, dnext)), _gspec(nch, dcp)],
        out_shape=[
            jax.ShapeDtypeStruct((N, dout), jnp.float32),
            jax.ShapeDtypeStruct((N, dnext), jnp.float32),
            jax.ShapeDtypeStruct((nch, N, dcp), jnp.float32),
        ],
    )(*args)


def _tc_comb_proj(s, p, dis, d2, b, Wn, nch, dcp):
    dout = p.shape[1]
    dnext = Wn.shape[1]

    def kern(s_ref, p_ref, dis_ref, d2_ref, b_ref, wn_ref, h_ref, pn_ref,
             g_ref):
        agg = _agg_of(s_ref, dout)
        h = jnp.maximum(
            dis_ref[...] * agg + d2_ref[...] * p_ref[...] + b_ref[...], 0.0)
        h_ref[...] = h
        pn = _dot(h, wn_ref[...])
        pn_ref[...] = pn
        _write_g(g_ref, pn * dis_ref[...], nch, dcp)

    return pl.pallas_call(
        kern,
        grid=(GRID,),
        in_specs=[
            _sspec(s),
            _rows((BM, dout)), _rows((BM, 1)), _rows((BM, 1)),
            _full((1, dout)), _full((dout, dnext)),
        ],
        out_specs=[_rows((BM, dout)), _rows((BM, dnext)), _gspec(nch, dcp)],
        out_shape=[
            jax.ShapeDtypeStruct((N, dout), jnp.float32),
            jax.ShapeDtypeStruct((N, dnext), jnp.float32),
            jax.ShapeDtypeStruct((nch, N, dcp), jnp.float32),
        ],
    )(s, p, dis, d2, b, Wn)


def _tc_comb(s, p, dis, d2, b, g_out=None):
    dout = p.shape[1]

    def kern(s_ref, p_ref, dis_ref, d2_ref, b_ref, h_ref, *maybe_g):
        agg = _agg_of(s_ref, dout)
        h = jnp.maximum(
            dis_ref[...] * agg + d2_ref[...] * p_ref[...] + b_ref[...], 0.0)
        h_ref[...] = h
        if maybe_g:
            _write_g(maybe_g[0], h * dis_ref[...], *g_out)

    in_specs = [
        _sspec(s),
        _rows((BM, dout)), _rows((BM, 1)), _rows((BM, 1)), _full((1, dout)),
    ]
    out_specs = [_rows((BM, dout))]
    out_shape = [jax.ShapeDtypeStruct((N, dout), jnp.float32)]
    if g_out is not None:
        nch, dcp = g_out
        out_specs.append(_gspec(nch, dcp))
        out_shape.append(jax.ShapeDtypeStruct((nch, N, dcp), jnp.float32))
    res = pl.pallas_call(kern, grid=(GRID,), in_specs=in_specs,
                         out_specs=out_specs, out_shape=out_shape)(
                             s, p, dis, d2, b)
    return res if g_out is not None else (res[0],)


def _tc_mm_agg(s, hprev, dis, d2, W, b, skip=None, g_out=None):
    din, dout = W.shape

    def kern(*refs):
        it = iter(refs)
        s_ref, h_ref, dis_ref, d2_ref, w_ref, b_ref = (
            next(it), next(it), next(it), next(it), next(it), next(it))
        skip_ref = next(it) if skip is not None else None
        o_ref = next(it)
        g_ref = next(it) if g_out is not None else None
        u = dis_ref[...] * _agg_of(s_ref, din) + d2_ref[...] * h_ref[...]
        acc = jnp.dot(u, w_ref[...], preferred_element_type=jnp.float32,
                      precision=lax.Precision.DEFAULT) + b_ref[...]
        if skip_ref is not None:
            acc = acc + skip_ref[...]
        h = jnp.maximum(acc, 0.0)
        o_ref[...] = h
        if g_ref is not None:
            _write_g(g_ref, h * dis_ref[...], *g_out)

    in_specs = [
        _sspec(s),
        _rows((BM, din)), _rows((BM, 1)), _rows((BM, 1)),
        _full((din, dout)), _full((1, dout)),
    ]
    args = [s, hprev, dis, d2, W, b]
    if skip is not None:
        in_specs.append(_rows((BM, dout)))
        args.append(skip)
    out_specs = [_rows((BM, dout))]
    out_shape = [jax.ShapeDtypeStruct((N, dout), jnp.float32)]
    if g_out is not None:
        nch, dcp = g_out
        out_specs.append(_gspec(nch, dcp))
        out_shape.append(jax.ShapeDtypeStruct((nch, N, dcp), jnp.float32))
    res = pl.pallas_call(kern, grid=(GRID,), in_specs=in_specs,
                         out_specs=out_specs, out_shape=out_shape)(*args)
    return res if g_out is not None else (res[0],)


@jax.jit
def _impl(x, edge_index, Ws, bs):
    src = _blocked_idx(edge_index[0])
    dst = _blocked_idx(edge_index[1])
    b2d = [b.reshape(1, -1) for b in bs]

    ones16 = jnp.ones((1, N, 16), jnp.float32)
    degp = _sc_run(ones16, src, dst)
    dis, d2, g0 = _tc_deg(degp, x)

    s = _sc_run(g0, src, dst)
    h1, p1, g1 = _tc_agg_proj(s, x, dis, d2, Ws[0], b2d[0], Ws[1],
                              *_chunking(320))

    s = _sc_run(g1, src, dst)
    h2, p2, g2 = _tc_comb_proj(s, p1, dis, d2, b2d[1], Ws[2], *_chunking(160))
    s = _sc_run(g2, src, dst)
    h3, p3, g3 = _tc_comb_proj(s, p2, dis, d2, b2d[2], Ws[3], *_chunking(80))
    s = _sc_run(g3, src, dst)
    h4, p4, g4 = _tc_comb_proj(s, p3, dis, d2, b2d[3], Ws[4], *_chunking(40))

    s = _sc_run(g4, src, dst)
    h5, g5 = _tc_comb(s, p4, dis, d2, b2d[4], g_out=(1, 48))

    s = _sc_run(g5, src, dst)
    h6, g6 = _tc_mm_agg(s, h5, dis, d2, Ws[5], b2d[5], skip=h4,
                        g_out=_chunking(80))
    s = _sc_run(g6, src, dst)
    h7, g7 = _tc_mm_agg(s, h6, dis, d2, Ws[6], b2d[6], skip=h3,
                        g_out=_chunking(160))
    s = _sc_run(g7, src, dst)
    h8, g8 = _tc_mm_agg(s, h7, dis, d2, Ws[7], b2d[7], skip=h2,
                        g_out=_chunking(320))

    s = _sc_run(g8, src, dst)
    _h9, p9, g9 = _tc_agg_proj(s, h8, dis, d2, Ws[8], b2d[8], Ws[9],
                               *_chunking(128), skip=h1)

    s = _sc_run(g9, src, dst)
    (out,) = _tc_comb(s, p9, dis, d2, b2d[9])
    return out


def kernel(x, edge_index, Ws, bs):
    return _impl(x, edge_index, list(Ws), list(bs))
